# baseline (speedup 1.0000x reference)
import functools

import jax
import jax.numpy as jnp
from jax import lax
from jax.experimental import pallas as pl
from jax.experimental.pallas import tpu as pltpu
from jax.experimental.pallas import tpu_sc as plsc


def _build_gather(n_chunks: int, chunk: int, d: int, nc: int, ns: int):
    nw = nc * ns
    mesh = plsc.VectorSubcoreMesh(core_axis_name="c", subcore_axis_name="s")

    @functools.partial(
        pl.kernel,
        mesh=mesh,
        compiler_params=pltpu.CompilerParams(use_tc_tiling_on_sc=False),
        out_type=jax.ShapeDtypeStruct((nw, n_chunks, chunk, d), jnp.float32),
        scratch_types=[
            pltpu.VMEM((n_chunks, chunk), jnp.int32),
            pltpu.VMEM((chunk, d), jnp.float32),
            pltpu.SemaphoreType.DMA,
        ],
    )
    def gather_kernel(idx_hbm, table_hbm, out_hbm, idx_v, rows_v, sem):
        wid = lax.axis_index("s") * nc + lax.axis_index("c")
        pltpu.sync_copy(idx_hbm.at[wid], idx_v)

        def step(j, carry):
            pltpu.async_copy(table_hbm.at[idx_v.at[j]], rows_v, sem).wait()
            pltpu.sync_copy(rows_v, out_hbm.at[wid, j])
            return carry

        lax.fori_loop(0, n_chunks, step, 0)

    return gather_kernel


def kernel(text, table):
    b, l = text.shape
    v, d = table.shape
    n = b * l

    info = plsc.get_sparse_core_info()
    nc, ns = info.num_cores, info.num_subcores
    nw = nc * ns

    chunk = 128
    assert n % (nw * chunk) == 0
    n_chunks = n // (nw * chunk)

    idx = text.reshape(nw, n_chunks, chunk)
    gather_kernel = _build_gather(n_chunks, chunk, d, nc, ns)
    out = gather_kernel(idx, table)
    return out.reshape(b, l, d)


# speedup vs baseline: 1.0474x; 1.0474x over previous
import functools

import jax
import jax.numpy as jnp
from jax import lax
from jax.experimental import pallas as pl
from jax.experimental.pallas import tpu as pltpu
from jax.experimental.pallas import tpu_sc as plsc


def _build_gather(n_chunks: int, chunk: int, nbuf: int, d: int, nc: int, ns: int):
    nw = nc * ns
    s = nbuf * chunk
    g_total = n_chunks // nbuf
    n_per_w = n_chunks * chunk
    mesh = plsc.VectorSubcoreMesh(core_axis_name="c", subcore_axis_name="s")

    @functools.partial(
        pl.kernel,
        mesh=mesh,
        compiler_params=pltpu.CompilerParams(use_tc_tiling_on_sc=False),
        out_type=jax.ShapeDtypeStruct((nw, n_per_w, d), jnp.float32),
        scratch_types=[
            pltpu.VMEM((n_chunks, chunk), jnp.int32),
            pltpu.VMEM((2, s, d), jnp.float32),
            pltpu.SemaphoreType.DMA,
            pltpu.SemaphoreType.DMA,
        ],
    )
    def gather_kernel(idx_hbm, table_hbm, out_hbm, idx_v, rows_v, gsem, ssem):
        wid = lax.axis_index("s") * nc + lax.axis_index("c")
        pltpu.sync_copy(idx_hbm.at[wid], idx_v)

        def fire_group(g, slot):
            for b in range(nbuf):
                pltpu.async_copy(
                    table_hbm.at[idx_v.at[g * nbuf + b]],
                    rows_v.at[slot, pl.ds(b * chunk, chunk)],
                    gsem,
                )

        def drain_gathers(slot):
            pltpu.make_async_copy(
                table_hbm.at[pl.ds(0, s)], rows_v.at[slot], gsem
            ).wait()

        def fire_store(g, slot):
            pltpu.async_copy(
                rows_v.at[slot], out_hbm.at[wid, pl.ds(g * s, s)], ssem
            )

        def drain_store():
            pltpu.make_async_copy(
                rows_v.at[0], out_hbm.at[wid, pl.ds(0, s)], ssem
            ).wait()

        fire_group(0, 0)
        drain_gathers(0)
        fire_group(1, 1)
        fire_store(0, 0)

        def step(g, carry):
            slot = lax.rem(g, 2)
            drain_gathers(slot)
            drain_store()
            fire_group(g + 1, 1 - slot)
            fire_store(g, slot)
            return carry

        lax.fori_loop(1, g_total - 1, step, 0)

        last = g_total - 1
        drain_gathers(lax.rem(last, 2))
        drain_store()
        fire_store(last, lax.rem(last, 2))
        drain_store()

    return gather_kernel


def kernel(text, table):
    b, l = text.shape
    v, d = table.shape
    n = b * l

    info = plsc.get_sparse_core_info()
    nc, ns = info.num_cores, info.num_subcores
    nw = nc * ns

    chunk = 128
    nbuf = 5
    assert n % (nw * chunk) == 0
    n_chunks = n // (nw * chunk)
    assert n_chunks % nbuf == 0 and n_chunks // nbuf >= 2

    idx = text.reshape(nw, n_chunks, chunk)
    gather_kernel = _build_gather(n_chunks, chunk, nbuf, d, nc, ns)
    out = gather_kernel(idx, table)
    return out.reshape(b, l, d)
